# baseline (speedup 1.0000x reference)
import functools

import jax
import jax.numpy as jnp
from jax import lax
from jax.experimental import pallas as pl
from jax.experimental.pallas import tpu as pltpu
from jax.experimental.pallas import tpu_sc as plsc


def _make_sc_kernel(B, S, D):
    info = plsc.get_sparse_core_info()
    NC, NS = info.num_cores, info.num_subcores
    NW = NC * NS
    RPW = S // NW
    CH = 64
    NCH = RPW // CH
    mesh = plsc.VectorSubcoreMesh(core_axis_name="c", subcore_axis_name="s")

    @functools.partial(
        pl.kernel,
        out_type=jax.ShapeDtypeStruct((B, S, D), jnp.float32),
        mesh=mesh,
        scratch_types=[
            pltpu.VMEM((CH, D), jnp.float32),
            pltpu.SemaphoreType.DMA,
        ],
    )
    def k(w_hbm, out_hbm, buf, sem):
        wid = lax.axis_index("s") * NC + lax.axis_index("c")
        base = wid * RPW
        for c in range(NCH):
            start = base + c * CH
            pltpu.sync_copy(w_hbm.at[pl.ds(start, CH), :], buf)
            for b in range(B):
                pltpu.sync_copy(buf, out_hbm.at[b, pl.ds(start, CH), :])

    return k


def kernel(x, W):
    B, S = x.shape
    D = W.shape[1]
    assert S % 2048 == 0
    return _make_sc_kernel(B, S, D)(W[:S])
